# baseline (speedup 1.0000x reference)
import functools

import jax
import jax.numpy as jnp
from jax.experimental import pallas as pl
from jax.experimental.pallas import tpu as pltpu


def _body(x_ref, l_ref, w_ref, bv_ref, fc_ref, out_ref, p_ref, a_ref, *,
          kk, filt, feat, np_, tb):
    @pl.when(pl.program_id(0) == 0)
    def _build_ops():
        lv = l_ref[...]
        r = jax.lax.broadcasted_iota(jnp.int32, (np_, np_), 0)
        c = jax.lax.broadcasted_iota(jnp.int32, (np_, np_), 1)
        t0 = (r == c).astype(jnp.float32)
        p_ref[0, :, :] = t0
        p_ref[1, :, :] = lv
        t1 = lv
        for k in range(2, kk):
            t2 = 2.0 * jax.lax.dot(lv, t1, precision=jax.lax.Precision.HIGHEST,
                                   preferred_element_type=jnp.float32) - t0
            p_ref[k, :, :] = t2
            t0, t1 = t1, t2

        bmask = (jax.lax.broadcasted_iota(jnp.int32, (np_, np_), 1)
                 == np_ - 1).astype(jnp.float32)

        def abody(f, carry):
            acc = p_ref[0, :, :] * w_ref[0, f]
            for k in range(1, kk):
                acc = acc + p_ref[k, :, :] * w_ref[k, f]
            acc = acc + bmask * bv_ref[f]
            a_ref[pl.ds(f * np_, np_), :] = acc.astype(jnp.bfloat16)
            return carry

        jax.lax.fori_loop(0, filt, abody, 0)

    av = a_ref[...]
    acc = jnp.zeros((8, tb), jnp.float32)
    for s in range(feat):
        xs = x_ref[s, :, :].astype(jnp.bfloat16)
        g = jax.lax.dot(av, xs, preferred_element_type=jnp.float32)
        g = jnp.maximum(g, 0.0)
        acc = acc + jax.lax.dot(fc_ref[s, :, :], g,
                                preferred_element_type=jnp.float32)
    out_ref[...] = acc


def kernel(x, L, W, b, fc_w, fc_b, y):
    B, N, F = x.shape
    K, FILT = W.shape
    C = fc_w.shape[1]
    NP = 64
    TB = 1024
    AROWS = FILT * NP

    x5 = jnp.pad(x.transpose(2, 1, 0), ((0, 0), (0, NP - N), (0, 0)))
    x5 = x5.at[:, NP - 1, :].set(1.0)
    Lp = jnp.pad(L, ((0, NP - N), (0, NP - N)))
    bvec = b.reshape(FILT)
    fc5 = jnp.pad(fc_w.reshape(FILT, N, F, C).transpose(2, 3, 0, 1),
                  ((0, 0), (0, 8 - C), (0, 0), (0, NP - N)))
    fcT = fc5.reshape(F, 8, AROWS)

    body = functools.partial(_body, kk=K, filt=FILT, feat=F, np_=NP, tb=TB)
    out = pl.pallas_call(
        body,
        grid=(B // TB,),
        in_specs=[
            pl.BlockSpec((F, NP, TB), lambda i: (0, 0, i)),
            pl.BlockSpec((NP, NP), lambda i: (0, 0)),
            pl.BlockSpec(memory_space=pltpu.SMEM),
            pl.BlockSpec(memory_space=pltpu.SMEM),
            pl.BlockSpec((F, 8, AROWS), lambda i: (0, 0, 0)),
        ],
        out_specs=pl.BlockSpec((8, TB), lambda i: (0, i)),
        out_shape=jax.ShapeDtypeStruct((8, B), jnp.float32),
        scratch_shapes=[
            pltpu.VMEM((K, NP, NP), jnp.float32),
            pltpu.VMEM((AROWS, NP), jnp.bfloat16),
        ],
        compiler_params=pltpu.CompilerParams(
            dimension_semantics=("arbitrary",)),
    )(x5, Lp, W, bvec, fcT)
    return out[:C, :].T + fc_b[None, :]


# speedup vs baseline: 1.1107x; 1.1107x over previous
import functools

import jax
import jax.numpy as jnp
from jax.experimental import pallas as pl
from jax.experimental.pallas import tpu as pltpu


def _body(x_ref, l_ref, w_ref, bv_ref, fc_ref, out_ref, p_ref, a_ref, *,
          kk, filt, feat, np_, tb):
    @pl.when(pl.program_id(0) == 0)
    def _build_ops():
        lv = l_ref[...]
        r = jax.lax.broadcasted_iota(jnp.int32, (np_, np_), 0)
        c = jax.lax.broadcasted_iota(jnp.int32, (np_, np_), 1)
        t0 = (r == c).astype(jnp.float32)
        p_ref[0, :, :] = t0
        p_ref[1, :, :] = lv
        t1 = lv
        for k in range(2, kk):
            t2 = 2.0 * jax.lax.dot(lv, t1, precision=jax.lax.Precision.HIGHEST,
                                   preferred_element_type=jnp.float32) - t0
            p_ref[k, :, :] = t2
            t0, t1 = t1, t2

        bmask = (jax.lax.broadcasted_iota(jnp.int32, (np_, np_), 1)
                 == np_ - 1).astype(jnp.float32)

        def abody(f, carry):
            acc = p_ref[0, :, :] * w_ref[0, f]
            for k in range(1, kk):
                acc = acc + p_ref[k, :, :] * w_ref[k, f]
            acc = acc + bmask * bv_ref[f]
            a_ref[pl.ds(f * np_, np_), :] = acc.astype(jnp.bfloat16)
            return carry

        jax.lax.fori_loop(0, filt, abody, 0)

    av = a_ref[...]
    acc = jnp.zeros((8, tb), jnp.float32)
    for s in range(feat):
        xs = x_ref[s, :, :].astype(jnp.bfloat16)
        g = jax.lax.dot(av, xs, preferred_element_type=jnp.float32)
        g = jnp.maximum(g, 0.0).astype(jnp.bfloat16)
        acc = acc + jax.lax.dot(fc_ref[s, :, :], g,
                                preferred_element_type=jnp.float32)
    out_ref[...] = acc


def kernel(x, L, W, b, fc_w, fc_b, y):
    B, N, F = x.shape
    K, FILT = W.shape
    C = fc_w.shape[1]
    NP = 64
    TB = 1024
    AROWS = FILT * NP

    x5 = jnp.pad(x.transpose(2, 1, 0), ((0, 0), (0, NP - N), (0, 0)))
    x5 = x5.at[:, NP - 1, :].set(1.0)
    Lp = jnp.pad(L, ((0, NP - N), (0, NP - N)))
    bvec = b.reshape(FILT)
    fc5 = jnp.pad(fc_w.reshape(FILT, N, F, C).transpose(2, 3, 0, 1),
                  ((0, 0), (0, 8 - C), (0, 0), (0, NP - N)))
    fcT = fc5.reshape(F, 8, AROWS).astype(jnp.bfloat16)

    body = functools.partial(_body, kk=K, filt=FILT, feat=F, np_=NP, tb=TB)
    out = pl.pallas_call(
        body,
        grid=(B // TB,),
        in_specs=[
            pl.BlockSpec((F, NP, TB), lambda i: (0, 0, i)),
            pl.BlockSpec((NP, NP), lambda i: (0, 0)),
            pl.BlockSpec(memory_space=pltpu.SMEM),
            pl.BlockSpec(memory_space=pltpu.SMEM),
            pl.BlockSpec((F, 8, AROWS), lambda i: (0, 0, 0)),
        ],
        out_specs=pl.BlockSpec((8, TB), lambda i: (0, i)),
        out_shape=jax.ShapeDtypeStruct((8, B), jnp.float32),
        scratch_shapes=[
            pltpu.VMEM((K, NP, NP), jnp.float32),
            pltpu.VMEM((AROWS, NP), jnp.bfloat16),
        ],
        compiler_params=pltpu.CompilerParams(
            dimension_semantics=("arbitrary",)),
    )(x5, Lp, W, bvec, fcT)
    return out[:C, :].T + fc_b[None, :]
